# baseline (speedup 1.0000x reference)
import functools

import jax
import jax.numpy as jnp
from jax import lax
from jax.experimental import pallas as pl
from jax.experimental.pallas import tpu as pltpu
from jax.experimental.pallas import tpu_sc as plsc


def _node_body(x_ref, w_ref, out_ref):
    d = x_ref.shape[1]
    out_ref[0:1] = lax.dot_general(
        w_ref[:, :d], x_ref[...],
        (((1,), (1,)), ((), ())),
        preferred_element_type=jnp.float32,
    )
    out_ref[1:2] = lax.dot_general(
        w_ref[:, d : 2 * d], x_ref[...],
        (((1,), (1,)), ((), ())),
        preferred_element_type=jnp.float32,
    )


def _node_scores(x, W):
    n, d = x.shape
    return pl.pallas_call(
        _node_body,
        out_shape=jax.ShapeDtypeStruct((2, n), jnp.float32),
    )(x, W)


def _make_gather(n_nodes, n_edges):
    info = plsc.get_sparse_core_info()
    nc, ns = info.num_cores, info.num_subcores
    nc = 1
    nw = nc * ns
    assert n_edges % (nw * 16) == 0
    chunk = n_edges // nw
    mesh = plsc.VectorSubcoreMesh(
        core_axis_name="c", subcore_axis_name="s", num_cores=1)

    @functools.partial(
        pl.kernel,
        mesh=mesh,
        out_type=jax.ShapeDtypeStruct((n_edges,), jnp.float32),
        compiler_params=pltpu.CompilerParams(needs_layout_passes=False),
        cost_estimate=pl.CostEstimate(
            flops=2 * n_edges,
            transcendentals=0,
            bytes_accessed=16 * n_edges,
        ),
        scratch_types=[
            pltpu.VMEM((chunk,), jnp.int32),
            pltpu.VMEM((chunk,), jnp.int32),
            pltpu.VMEM((2 * n_nodes,), jnp.float32),
            pltpu.VMEM((chunk,), jnp.float32),
            pltpu.SemaphoreType.DMA,
            pltpu.SemaphoreType.DMA,
            pltpu.SemaphoreType.DMA,
            pltpu.SemaphoreType.DMA,
        ],
    )
    def gather_kernel(s2_hbm, eidx_hbm, dote_hbm, out_hbm, src_v, dst_v,
                      tbl_v, out_v, sem0, sem1, sem2, sem3):
        wid = lax.axis_index("s") * nc + lax.axis_index("c")
        base = wid * chunk
        cp0 = pltpu.async_copy(eidx_hbm.at[pl.ds(base, chunk)], src_v, sem0)
        cp1 = pltpu.async_copy(
            eidx_hbm.at[pl.ds(n_edges + base, chunk)], dst_v, sem1)
        cp2 = pltpu.async_copy(s2_hbm, tbl_v, sem2)
        cp3 = pltpu.async_copy(dote_hbm.at[pl.ds(base, chunk)], out_v, sem3)
        cp0.wait()
        cp1.wait()
        cp2.wait()
        cp3.wait()

        @plsc.parallel_loop(0, chunk, 16, unroll=8)
        def body(i):
            sv = src_v[pl.ds(i, 16)]
            dv = dst_v[pl.ds(i, 16)]
            a = plsc.load_gather(tbl_v, [sv])
            bb = plsc.load_gather(tbl_v, [dv + n_nodes])
            out_v[pl.ds(i, 16)] = out_v[pl.ds(i, 16)] + a + bb

        pltpu.sync_copy(out_v, out_hbm.at[pl.ds(base, chunk)])

    return gather_kernel


def _edge_body(e_ref, w_ref, b_ref, out_ref):
    d = e_ref.shape[1]
    s = lax.dot_general(
        w_ref[:, 2 * d :], e_ref[...],
        (((1,), (1,)), ((), ())),
        preferred_element_type=jnp.float32,
    )
    out_ref[0] = s + b_ref[0]


def _edge_scores(e, we, b, nb, be):
    n_edges, d = e.shape
    assert nb * be == n_edges
    return pl.pallas_call(
        _edge_body,
        grid=(nb,),
        in_specs=[
            pl.BlockSpec((be, d), lambda i: (i, 0)),
            pl.BlockSpec((1, 3 * d), lambda i: (0, 0)),
            pl.BlockSpec(memory_space=pltpu.SMEM),
        ],
        out_specs=pl.BlockSpec((1, 1, be), lambda i: (i, 0, 0)),
        out_shape=jax.ShapeDtypeStruct((nb, 1, be), jnp.float32),
    )(e, we, b)


def _add_body(a_ref, b_ref, out_ref):
    out_ref[...] = a_ref[...] + b_ref[...]


def _final_add(a, g2):
    nb, _, be = a.shape
    return pl.pallas_call(
        _add_body,
        grid=(nb,),
        in_specs=[
            pl.BlockSpec((1, 1, be), lambda i: (i, 0, 0)),
            pl.BlockSpec((1, 1, be), lambda i: (i, 0, 0)),
        ],
        out_specs=pl.BlockSpec((1, 1, be), lambda i: (i, 0, 0)),
        out_shape=jax.ShapeDtypeStruct((nb, 1, be), jnp.float32),
    )(a, g2)


def kernel(x, edge_index, e, W, b):
    n, d = x.shape
    n_edges = e.shape[0]
    be = 32000
    nb = n_edges // be
    s2 = _node_scores(x, W)
    dot_e = _edge_scores(e, W, b, nb, be)
    out = _make_gather(n, n_edges)(
        s2.reshape(2 * n), edge_index.reshape(2 * n_edges),
        dot_e.reshape(n_edges),
    )
    return out.reshape(n_edges, 1)


# speedup vs baseline: 1.1239x; 1.1239x over previous
import functools

import jax
import jax.numpy as jnp
from jax import lax
from jax.experimental import pallas as pl
from jax.experimental.pallas import tpu as pltpu
from jax.experimental.pallas import tpu_sc as plsc


def _node_body(x_ref, w_ref, out_ref):
    d = x_ref.shape[1]
    out_ref[0:1] = lax.dot_general(
        w_ref[:, :d], x_ref[...],
        (((1,), (1,)), ((), ())),
        preferred_element_type=jnp.float32,
    )
    out_ref[1:2] = lax.dot_general(
        w_ref[:, d : 2 * d], x_ref[...],
        (((1,), (1,)), ((), ())),
        preferred_element_type=jnp.float32,
    )


def _node_scores(x, W):
    n, d = x.shape
    return pl.pallas_call(
        _node_body,
        out_shape=jax.ShapeDtypeStruct((2, n), jnp.float32),
    )(x, W)


def _make_gather(n_nodes, n_edges):
    info = plsc.get_sparse_core_info()
    nc, ns = info.num_cores, info.num_subcores
    nc = 1
    nw = nc * ns
    assert n_edges % (nw * 16) == 0
    chunk = n_edges // nw
    mesh = plsc.VectorSubcoreMesh(
        core_axis_name="c", subcore_axis_name="s", num_cores=1)

    @functools.partial(
        pl.kernel,
        mesh=mesh,
        out_type=jax.ShapeDtypeStruct((n_edges,), jnp.float32),
        compiler_params=pltpu.CompilerParams(needs_layout_passes=False),
        cost_estimate=pl.CostEstimate(
            flops=2 * n_edges,
            transcendentals=0,
            bytes_accessed=16 * n_edges,
        ),
        scratch_types=[
            pltpu.VMEM((chunk,), jnp.int32),
            pltpu.VMEM((chunk,), jnp.int32),
            pltpu.VMEM((2 * n_nodes,), jnp.float32),
            pltpu.VMEM((chunk,), jnp.float32),
            pltpu.SemaphoreType.DMA,
            pltpu.SemaphoreType.DMA,
            pltpu.SemaphoreType.DMA,
        ],
    )
    def gather_kernel(s2_hbm, eidx_hbm, out_hbm, src_v, dst_v, tbl_v,
                      out_v, sem0, sem1, sem2):
        wid = lax.axis_index("s") * nc + lax.axis_index("c")
        base = wid * chunk
        cp0 = pltpu.async_copy(eidx_hbm.at[pl.ds(base, chunk)], src_v, sem0)
        cp1 = pltpu.async_copy(
            eidx_hbm.at[pl.ds(n_edges + base, chunk)], dst_v, sem1)
        cp2 = pltpu.async_copy(s2_hbm, tbl_v, sem2)
        cp0.wait()
        cp1.wait()
        cp2.wait()

        @plsc.parallel_loop(0, chunk, 16, unroll=8)
        def body(i):
            sv = src_v[pl.ds(i, 16)]
            dv = dst_v[pl.ds(i, 16)]
            a = plsc.load_gather(tbl_v, [sv])
            bb = plsc.load_gather(tbl_v, [dv + n_nodes])
            out_v[pl.ds(i, 16)] = a + bb

        pltpu.sync_copy(out_v, out_hbm.at[pl.ds(base, chunk)])

    return gather_kernel


def _edge_body(e_ref, w_ref, b_ref, out_ref):
    d = e_ref.shape[1]
    s = lax.dot_general(
        w_ref[:, 2 * d :], e_ref[...],
        (((1,), (1,)), ((), ())),
        preferred_element_type=jnp.float32,
    )
    out_ref[0] = s + b_ref[0]


def _edge_scores(e, we, b, nb, be):
    n_edges, d = e.shape
    assert nb * be == n_edges
    return pl.pallas_call(
        _edge_body,
        grid=(nb,),
        in_specs=[
            pl.BlockSpec((be, d), lambda i: (i, 0)),
            pl.BlockSpec((1, 3 * d), lambda i: (0, 0)),
            pl.BlockSpec(memory_space=pltpu.SMEM),
        ],
        out_specs=pl.BlockSpec((1, 1, be), lambda i: (i, 0, 0)),
        out_shape=jax.ShapeDtypeStruct((nb, 1, be), jnp.float32),
    )(e, we, b)


def _add_body(a_ref, b_ref, out_ref):
    out_ref[...] = a_ref[...] + b_ref[...]


def _final_add(a, g2):
    nb, _, be = a.shape
    return pl.pallas_call(
        _add_body,
        grid=(nb,),
        in_specs=[
            pl.BlockSpec((1, 1, be), lambda i: (i, 0, 0)),
            pl.BlockSpec((1, 1, be), lambda i: (i, 0, 0)),
        ],
        out_specs=pl.BlockSpec((1, 1, be), lambda i: (i, 0, 0)),
        out_shape=jax.ShapeDtypeStruct((nb, 1, be), jnp.float32),
    )(a, g2)


def kernel(x, edge_index, e, W, b):
    n, d = x.shape
    n_edges = e.shape[0]
    be = 32000
    nb = n_edges // be
    s2 = _node_scores(x, W)
    dot_e = _edge_scores(e, W, b, nb, be)
    g = _make_gather(n, n_edges)(
        s2.reshape(2 * n), edge_index.reshape(2 * n_edges)
    )
    out = _final_add(dot_e, g.reshape(nb, 1, be))
    return out.reshape(n_edges, 1)
